# Initial kernel scaffold; baseline (speedup 1.0000x reference)
import jax
import jax.numpy as jnp
from jax import lax
from jax.experimental import pallas as pl
from jax.experimental.pallas import tpu as pltpu
from jax.experimental.pallas import tpu_sc as plsc

B = 16384
S = 50
D = 32
NC = 2
NS = 16
NW = NC * NS
PW = B // NW
CB = 64
NCHUNK = PW // CB
L = 16


def _body(ids_hbm, table_hbm, out_hbm, idx_v, rows_v, acc_v, sem):
    wid = lax.axis_index("s") * NC + lax.axis_index("c")
    inv = jnp.float32(1.0 / S)

    def chunk_body(c, carry):
        base_rows = wid * PW + c * CB
        pltpu.sync_copy(ids_hbm.at[pl.ds(base_rows * S, CB * S)], idx_v)
        pltpu.async_copy(table_hbm.at[idx_v], rows_v, sem).wait()

        def row_body(g, carry2):
            r = g * S
            a0 = rows_v[r, pl.ds(0, L)]
            a1 = rows_v[r, pl.ds(L, L)]
            for s in range(1, S):
                a0 = a0 + rows_v[r + s, pl.ds(0, L)]
                a1 = a1 + rows_v[r + s, pl.ds(L, L)]
            acc_v[g, pl.ds(0, L)] = a0 * inv
            acc_v[g, pl.ds(L, L)] = a1 * inv
            return carry2

        lax.fori_loop(0, CB, row_body, 0)
        pltpu.sync_copy(acc_v, out_hbm.at[pl.ds(base_rows, CB)])
        return carry

    lax.fori_loop(0, NCHUNK, chunk_body, 0)


@jax.jit
def _encode(ids_flat, table):
    mesh = plsc.VectorSubcoreMesh(core_axis_name="c", subcore_axis_name="s")
    return pl.kernel(
        _body,
        out_type=jax.ShapeDtypeStruct((B, D), jnp.float32),
        mesh=mesh,
        scratch_types=[
            pltpu.VMEM((CB * S,), jnp.int32),
            pltpu.VMEM((CB * S, D), jnp.float32),
            pltpu.VMEM((CB, D), jnp.float32),
            pltpu.SemaphoreType.DMA,
        ],
    )(ids_flat, table)


def kernel(prompt_ids, table):
    ids_flat = prompt_ids.astype(jnp.int32).reshape(B * S)
    return _encode(ids_flat, table)


# speedup vs baseline: 2.8044x; 2.8044x over previous
import jax
import jax.numpy as jnp
from jax import lax
from jax.experimental import pallas as pl
from jax.experimental.pallas import tpu as pltpu
from jax.experimental.pallas import tpu_sc as plsc

B = 16384
S = 50
D = 32
NC = 2
NS = 16
NW = NC * NS
PW = B // NW
CB = 64
NCHUNK = PW // CB
L = 16


def _body(ids_hbm, table_hbm, out_hbm, idx_v, rows_v, acc_v, sem):
    wid = lax.axis_index("s") * NC + lax.axis_index("c")
    inv = jnp.float32(1.0 / S)

    def chunk_body(c, carry):
        base_rows = wid * PW + c * CB
        pltpu.sync_copy(ids_hbm.at[pl.ds(base_rows * S, CB * S)], idx_v)
        pltpu.async_copy(table_hbm.at[idx_v], rows_v, sem).wait()

        def row_body(g, carry2):
            r = g * S
            a0 = rows_v[r, pl.ds(0, L)]
            a1 = rows_v[r, pl.ds(L, L)]
            for s in range(1, S):
                a0 = a0 + rows_v[r + s, pl.ds(0, L)]
                a1 = a1 + rows_v[r + s, pl.ds(L, L)]
            acc_v[g, pl.ds(0, L)] = a0 * inv
            acc_v[g, pl.ds(L, L)] = a1 * inv
            return carry2

        lax.fori_loop(0, CB, row_body, 0)
        pltpu.sync_copy(acc_v, out_hbm.at[pl.ds(base_rows, CB)])
        return carry

    lax.fori_loop(0, NCHUNK, chunk_body, 0)


@jax.jit
def _encode(ids_flat, table):
    mesh = plsc.VectorSubcoreMesh(core_axis_name="c", subcore_axis_name="s")
    return pl.kernel(
        _body,
        out_type=jax.ShapeDtypeStruct((B, D), jnp.float32),
        mesh=mesh,
        scratch_types=[
            pltpu.VMEM((CB * S,), jnp.int32),
            pltpu.VMEM((CB * S, D), jnp.float32),
            pltpu.VMEM((CB, D), jnp.float32),
            pltpu.SemaphoreType.DMA,
        ],
        compiler_params=pltpu.CompilerParams(use_tc_tiling_on_sc=False),
    )(ids_flat, table)


def kernel(prompt_ids, table):
    ids_flat = prompt_ids.astype(jnp.int32).reshape(B * S)
    return _encode(ids_flat, table)


# speedup vs baseline: 2.9397x; 1.0483x over previous
import jax
import jax.numpy as jnp
from jax import lax
from jax.experimental import pallas as pl
from jax.experimental.pallas import tpu as pltpu
from jax.experimental.pallas import tpu_sc as plsc

B = 16384
S = 50
D = 32
NC = 2
NS = 16
NW = NC * NS
PW = B // NW
CB = 32
NCHUNK = PW // CB
NPAIR = NCHUNK // 2
L = 16


def _body(ids_hbm, table_hbm, out_hbm,
          idx0, idx1, rows0, rows1, acc0, acc1, sem0, sem1):
    wid = lax.axis_index("s") * NC + lax.axis_index("c")
    inv = jnp.float32(1.0 / S)

    def stage_and_fire(c, idx_v, rows_v, sem):
        base_rows = wid * PW + c * CB
        pltpu.sync_copy(ids_hbm.at[pl.ds(base_rows * S, CB * S)], idx_v)
        pltpu.async_copy(table_hbm.at[idx_v], rows_v, sem)

    def pool(c, idx_v, rows_v, acc_v, sem):
        pltpu.make_async_copy(table_hbm.at[idx_v], rows_v, sem).wait()

        def row_body(g, carry2):
            r = g * S
            a0 = rows_v[r, pl.ds(0, L)]
            a1 = rows_v[r, pl.ds(L, L)]
            for s in range(1, S):
                a0 = a0 + rows_v[r + s, pl.ds(0, L)]
                a1 = a1 + rows_v[r + s, pl.ds(L, L)]
            acc_v[g, pl.ds(0, L)] = a0 * inv
            acc_v[g, pl.ds(L, L)] = a1 * inv
            return carry2

        lax.fori_loop(0, CB, row_body, 0)
        base_rows = wid * PW + c * CB
        pltpu.sync_copy(acc_v, out_hbm.at[pl.ds(base_rows, CB)])

    stage_and_fire(0, idx0, rows0, sem0)

    def pair_body(i, carry):
        c0 = 2 * i
        stage_and_fire(c0 + 1, idx1, rows1, sem1)
        pool(c0, idx0, rows0, acc0, sem0)

        @pl.when(i + 1 < NPAIR)
        def _():
            stage_and_fire(c0 + 2, idx0, rows0, sem0)

        pool(c0 + 1, idx1, rows1, acc1, sem1)
        return carry

    lax.fori_loop(0, NPAIR, pair_body, 0)


@jax.jit
def _encode(ids_flat, table):
    mesh = plsc.VectorSubcoreMesh(core_axis_name="c", subcore_axis_name="s")
    return pl.kernel(
        _body,
        out_type=jax.ShapeDtypeStruct((B, D), jnp.float32),
        mesh=mesh,
        scratch_types=[
            pltpu.VMEM((CB * S,), jnp.int32),
            pltpu.VMEM((CB * S,), jnp.int32),
            pltpu.VMEM((CB * S, D), jnp.float32),
            pltpu.VMEM((CB * S, D), jnp.float32),
            pltpu.VMEM((CB, D), jnp.float32),
            pltpu.VMEM((CB, D), jnp.float32),
            pltpu.SemaphoreType.DMA,
            pltpu.SemaphoreType.DMA,
        ],
        compiler_params=pltpu.CompilerParams(use_tc_tiling_on_sc=False),
    )(ids_flat, table)


def kernel(prompt_ids, table):
    ids_flat = prompt_ids.astype(jnp.int32).reshape(B * S)
    return _encode(ids_flat, table)
